# baseline (speedup 1.0000x reference)
import jax
import jax.numpy as jnp
from jax.experimental import pallas as pl
from jax.experimental.pallas import tpu as pltpu

_H = 16
_D = 64
_K = 1024
_CHUNK = 2048


def _vq_kernel(m_ref, x_ref, o_ref):
    a = -2.0 * m_ref[0]
    xc = x_ref[0]
    s = jax.lax.dot_general(
        a, xc, (((1,), (0,)), ((), ())),
        preferred_element_type=jnp.float32)
    m = m_ref[0]
    m_sq = jnp.sum(m * m, axis=1, keepdims=True)
    d = s + m_sq
    o_ref[0, 0, 0, :] = jnp.argmin(d, axis=0).astype(jnp.int32)


def kernel(x, means):
    b, n, feat = x.shape
    bn = b * n
    h, k, dim = means.shape
    xt = x.reshape(bn, h, dim).transpose(1, 2, 0)
    nc = bn // _CHUNK
    out = pl.pallas_call(
        _vq_kernel,
        grid=(_H, nc),
        in_specs=[
            pl.BlockSpec((1, _K, _D), lambda hh, c: (hh, 0, 0)),
            pl.BlockSpec((1, _D, _CHUNK), lambda hh, c: (hh, 0, c)),
        ],
        out_specs=pl.BlockSpec((1, 1, 1, _CHUNK), lambda hh, c: (hh, c, 0, 0)),
        out_shape=jax.ShapeDtypeStruct((_H, nc, 1, _CHUNK), jnp.int32),
        compiler_params=pltpu.CompilerParams(
            dimension_semantics=("parallel", "parallel")),
    )(means, xt)
    return out.reshape(_H, bn).T.reshape(b, n, _H)


# speedup vs baseline: 1.0452x; 1.0452x over previous
import jax
import jax.numpy as jnp
from jax.experimental import pallas as pl
from jax.experimental.pallas import tpu as pltpu

_H = 16
_D = 64
_K = 1024
_CHUNK = 4096


def _vq_kernel(m_ref, x_ref, o_ref):
    a = -2.0 * m_ref[0]
    xc = x_ref[0]
    s = jax.lax.dot_general(
        a, xc, (((1,), (0,)), ((), ())),
        preferred_element_type=jnp.float32)
    m = m_ref[0]
    m_sq = jnp.sum(m * m, axis=1, keepdims=True)
    d = s + m_sq
    o_ref[0, 0, 0, :] = jnp.argmin(d, axis=0).astype(jnp.int32)


def kernel(x, means):
    b, n, feat = x.shape
    bn = b * n
    h, k, dim = means.shape
    xt = x.reshape(bn, h, dim).transpose(1, 2, 0)
    nc = bn // _CHUNK
    out = pl.pallas_call(
        _vq_kernel,
        grid=(_H, nc),
        in_specs=[
            pl.BlockSpec((1, _K, _D), lambda hh, c: (hh, 0, 0)),
            pl.BlockSpec((1, _D, _CHUNK), lambda hh, c: (hh, 0, c)),
        ],
        out_specs=pl.BlockSpec((1, 1, 1, _CHUNK), lambda hh, c: (hh, c, 0, 0)),
        out_shape=jax.ShapeDtypeStruct((_H, nc, 1, _CHUNK), jnp.int32),
        compiler_params=pltpu.CompilerParams(
            dimension_semantics=("parallel", "parallel")),
    )(means, xt)
    return out.reshape(_H, bn).T.reshape(b, n, _H)
